# baseline (speedup 1.0000x reference)
import functools

import jax
import jax.numpy as jnp
from jax import lax
from jax.experimental import pallas as pl
from jax.experimental.pallas import tpu as pltpu
from jax.experimental.pallas import tpu_sc as plsc

N = 10000
E = 320000
D = 128
D2 = D // 2

NC = 2
NS = 16
NW = NC * NS
EPT = E // NS
G = 80
NGROUPS = EPT // G
SUB = 16
NSUB = G // SUB
NPAD = 10240
RPT = NPAD // NS
ZR = 128
LANES = 16


def _sc_segment_spmm(row3, col4, val3, x2):
    mesh = plsc.VectorSubcoreMesh(core_axis_name="c", subcore_axis_name="s")

    @functools.partial(
        pl.kernel,
        mesh=mesh,
        compiler_params=pltpu.CompilerParams(use_tc_tiling_on_sc=False),
        out_type=jax.ShapeDtypeStruct((NC * NPAD, D2), jnp.float32),
        scratch_types=[
            pltpu.VMEM((NGROUPS, G), jnp.int32),
            pltpu.VMEM((NGROUPS, G), jnp.int32),
            pltpu.VMEM((NGROUPS, G), jnp.float32),
            pltpu.VMEM((G, D2), jnp.float32),
            pltpu.VMEM((G, D2), jnp.float32),
            pltpu.VMEM((ZR, D2), jnp.float32),
            pltpu.VMEM_SHARED((NPAD, D2), jnp.float32),
            pltpu.SemaphoreType.DMA,
            pltpu.SemaphoreType.DMA,
        ],
    )
    def k(row_h, col_h, val_h, x_h, out_h, colv, rowv, valv, xr0, xr1,
          zbuf, agg, sem0, sem1):
        c = lax.axis_index("c")
        s = lax.axis_index("s")

        pltpu.sync_copy(col_h.at[c, s], colv)
        pltpu.sync_copy(val_h.at[s], valv)
        pltpu.sync_copy(row_h.at[s], rowv)

        def zb(i, carry):
            for kk in range(D2 // LANES):
                zbuf[i, pl.ds(LANES * kk, LANES)] = jnp.zeros(
                    (LANES,), jnp.float32)
            return carry
        lax.fori_loop(0, ZR, zb, 0)
        for j in range(RPT // ZR):
            pltpu.sync_copy(zbuf, agg.at[pl.ds(s * RPT + j * ZR, ZR)])
        plsc.subcore_barrier()

        def start_gather(g, xr, sem):
            pltpu.async_copy(x_h.at[colv.at[g]], xr, sem)

        def wait_gather(g, xr, sem):
            pltpu.make_async_copy(x_h.at[colv.at[g]], xr, sem).wait()

        dnums = lax.GatherDimensionNumbers(
            offset_dims=(), collapsed_slice_dims=(0,), start_index_map=(0,))

        def do_group(g, xr):
            def sub(t, carry2):
                v16 = valv[g, pl.ds(t * SUB, SUB)]
                for e in range(SUB):
                    idx = t * SUB + e
                    vv = lax.gather(
                        v16, jnp.full((LANES, 1), e, jnp.int32), dnums, (1,),
                        mode=lax.GatherScatterMode.PROMISE_IN_BOUNDS)
                    for kk in range(D2 // LANES):
                        sl = pl.ds(LANES * kk, LANES)
                        xr[idx, sl] = xr[idx, sl] * vv
                return carry2
            lax.fori_loop(0, NSUB, sub, 0)
            pltpu.sync_copy(xr, agg.at[rowv.at[g]], add=True)

        start_gather(0, xr0, sem0)

        def pair(i, carry):
            g0 = 2 * i
            wait_gather(g0, xr0, sem0)
            start_gather(g0 + 1, xr1, sem1)
            do_group(g0, xr0)
            wait_gather(g0 + 1, xr1, sem1)
            start_gather(g0 + 2, xr0, sem0)
            do_group(g0 + 1, xr1)
            return carry
        lax.fori_loop(0, NGROUPS // 2 - 1, pair, 0)
        wait_gather(NGROUPS - 2, xr0, sem0)
        start_gather(NGROUPS - 1, xr1, sem1)
        do_group(NGROUPS - 2, xr0)
        wait_gather(NGROUPS - 1, xr1, sem1)
        do_group(NGROUPS - 1, xr1)

        plsc.subcore_barrier()
        pltpu.sync_copy(agg.at[pl.ds(s * RPT, RPT)],
                        out_h.at[pl.ds(c * NPAD + s * RPT, RPT)])

    return k(row3, col4, val3, x2)


def _tc_body(p0, p1, wt0, wt1, bb, o):
    o[...] = (jnp.dot(p0[...], wt0[...], preferred_element_type=jnp.float32)
              + jnp.dot(p1[...], wt1[...],
                        preferred_element_type=jnp.float32)
              + bb[...])


def _tc_linear(partials, wt, b2):
    R = 512
    return pl.pallas_call(
        _tc_body,
        grid=(NPAD // R,),
        in_specs=[
            pl.BlockSpec((R, D2), lambda i: (i, 0)),
            pl.BlockSpec((R, D2), lambda i: (i + NPAD // R, 0)),
            pl.BlockSpec((D2, D), lambda i: (0, 0)),
            pl.BlockSpec((D2, D), lambda i: (1, 0)),
            pl.BlockSpec((1, D), lambda i: (0, 0)),
        ],
        out_specs=pl.BlockSpec((R, D), lambda i: (i, 0)),
        out_shape=jax.ShapeDtypeStruct((NPAD, D), jnp.float32),
    )(partials, partials, wt, wt, b2)


def kernel(L_indices, L_values, x, W, b):
    row3 = L_indices[0].reshape(NS, NGROUPS, G)
    col3 = L_indices[1].reshape(NS, NGROUPS, G)
    col4 = jnp.stack([col3, col3 + N])
    val3 = L_values.reshape(NS, NGROUPS, G)
    x2 = jnp.concatenate([x[:, :D2], x[:, D2:]], axis=0)
    pp = _sc_segment_spmm(row3, col4, val3, x2)
    out = _tc_linear(pp, W.T, b.reshape(1, D))
    return out[:N]
